# speedup vs baseline: 2.7491x; 2.7491x over previous
import functools

import jax
import jax.numpy as jnp
from jax import lax
from jax.experimental import pallas as pl
from jax.experimental.pallas import tpu as pltpu
from jax.experimental.pallas import tpu_sc as plsc

D_MODEL = 128
NUM_WORKERS = 32
CHUNK = 128


def _gather_sc(idx3, table, n_chunks):
    total = NUM_WORKERS * n_chunks * CHUNK
    mesh = plsc.VectorSubcoreMesh(core_axis_name="c", subcore_axis_name="s")

    @functools.partial(
        pl.kernel,
        mesh=mesh,
        out_type=jax.ShapeDtypeStruct((total, D_MODEL), jnp.float32),
        scratch_types=[
            pltpu.VMEM((n_chunks, CHUNK), jnp.int32),
            pltpu.VMEM((CHUNK, D_MODEL), jnp.float32),
            pltpu.SemaphoreType.DMA,
        ],
    )
    def k(idx_hbm, table_hbm, out_hbm, idx_v, rows_v, sem):
        wid = lax.axis_index("s") * 2 + lax.axis_index("c")
        base = wid * (n_chunks * CHUNK)
        pltpu.sync_copy(idx_hbm.at[wid], idx_v)

        def body(j, carry):
            pltpu.async_copy(table_hbm.at[idx_v.at[j]], rows_v, sem).wait()
            pltpu.sync_copy(rows_v, out_hbm.at[pl.ds(base + j * CHUNK, CHUNK)])
            return carry

        lax.fori_loop(0, n_chunks, body, 0)

    return k(idx3, table)


def kernel(indices, embedding):
    b, s = indices.shape
    total = b * s
    n_chunks = total // (NUM_WORKERS * CHUNK)
    idx3 = indices.astype(jnp.int32).reshape(NUM_WORKERS, n_chunks, CHUNK)
    out = _gather_sc(idx3, embedding, n_chunks)
    return out.reshape(b, s, D_MODEL)


# speedup vs baseline: 2.8436x; 1.0344x over previous
import functools

import jax
import jax.numpy as jnp
from jax import lax
from jax.experimental import pallas as pl
from jax.experimental.pallas import tpu as pltpu
from jax.experimental.pallas import tpu_sc as plsc

D_MODEL = 128
NUM_WORKERS = 32
CHUNK = 128


def _gather_sc(idx3, table, n_chunks):
    total = NUM_WORKERS * n_chunks * CHUNK
    mesh = plsc.VectorSubcoreMesh(core_axis_name="c", subcore_axis_name="s")

    @functools.partial(
        pl.kernel,
        mesh=mesh,
        out_type=jax.ShapeDtypeStruct((total, D_MODEL), jnp.float32),
        scratch_types=[
            pltpu.VMEM((n_chunks, CHUNK), jnp.int32),
            pltpu.VMEM((2, CHUNK, D_MODEL), jnp.float32),
            pltpu.SemaphoreType.DMA,
            pltpu.SemaphoreType.DMA,
        ],
    )
    def k(idx_hbm, table_hbm, out_hbm, idx_v, rows_v, gsem, wsem):
        wid = lax.axis_index("s") * 2 + lax.axis_index("c")
        base = wid * (n_chunks * CHUNK)
        pltpu.sync_copy(idx_hbm.at[wid], idx_v)

        def gather(j, b):
            pltpu.async_copy(table_hbm.at[idx_v.at[j]], rows_v.at[b], gsem)

        def write(j, b):
            pltpu.async_copy(
                rows_v.at[b], out_hbm.at[pl.ds(base + j * CHUNK, CHUNK)], wsem)

        def gwait(b):
            pltpu.make_async_copy(
                table_hbm.at[idx_v.at[0]], rows_v.at[b], gsem).wait()

        def wwait(b):
            pltpu.make_async_copy(
                rows_v.at[b], out_hbm.at[pl.ds(base, CHUNK)], wsem).wait()

        gather(0, 0)

        def body(j, carry):
            b = lax.rem(j, 2)
            gwait(b)

            @pl.when(j >= 1)
            def _():
                wwait(1 - b)

            @pl.when(j + 1 < n_chunks)
            def _():
                gather(j + 1, 1 - b)

            write(j, b)
            return carry

        lax.fori_loop(0, n_chunks, body, 0)
        wwait(lax.rem(n_chunks - 1, 2))

    return k(idx3, table)


def kernel(indices, embedding):
    b, s = indices.shape
    total = b * s
    n_chunks = total // (NUM_WORKERS * CHUNK)
    idx3 = indices.astype(jnp.int32).reshape(NUM_WORKERS, n_chunks, CHUNK)
    out = _gather_sc(idx3, embedding, n_chunks)
    return out.reshape(b, s, D_MODEL)


# speedup vs baseline: 4.7454x; 1.6688x over previous
import functools

import jax
import jax.numpy as jnp
from jax import lax
from jax.experimental import pallas as pl
from jax.experimental.pallas import tpu as pltpu
from jax.experimental.pallas import tpu_sc as plsc

D_MODEL = 128
NUM_WORKERS = 32
B_CHUNK = 8


def _gather_sc(idx3, table, b, s, s_pad):
    per_w = b // NUM_WORKERS
    n_sc = per_w // B_CHUNK
    mesh = plsc.VectorSubcoreMesh(core_axis_name="c", subcore_axis_name="s")

    @functools.partial(
        pl.kernel,
        mesh=mesh,
        out_type=jax.ShapeDtypeStruct((b, s, D_MODEL), jnp.float32),
        scratch_types=[
            pltpu.VMEM((per_w, s_pad), jnp.int32),
            pltpu.VMEM((2, B_CHUNK, s, D_MODEL), jnp.float32),
            pltpu.SemaphoreType.DMA,
            pltpu.SemaphoreType.DMA,
        ],
    )
    def k(idx_hbm, table_hbm, out_hbm, idx_v, rows_v, gsem, wsem):
        wid = lax.axis_index("s") * 2 + lax.axis_index("c")
        b0w = wid * per_w
        pltpu.sync_copy(idx_hbm.at[wid], idx_v)

        def gathers(j, buf):
            for i in range(B_CHUNK):
                pltpu.async_copy(
                    table_hbm.at[idx_v.at[j * B_CHUNK + i, pl.ds(0, s)]],
                    rows_v.at[buf, i], gsem)

        def gwait(buf):
            pltpu.make_async_copy(
                out_hbm.at[pl.ds(0, B_CHUNK)], rows_v.at[buf], gsem).wait()

        def write(j, buf):
            pltpu.async_copy(
                rows_v.at[buf], out_hbm.at[pl.ds(b0w + j * B_CHUNK, B_CHUNK)],
                wsem)

        def wwait(buf):
            pltpu.make_async_copy(
                rows_v.at[buf], out_hbm.at[pl.ds(b0w, B_CHUNK)], wsem).wait()

        gathers(0, 0)

        def body(j, carry):
            buf = lax.rem(j, 2)
            gwait(buf)

            @pl.when(j >= 1)
            def _():
                wwait(1 - buf)

            @pl.when(j + 1 < n_sc)
            def _():
                gathers(j + 1, 1 - buf)

            write(j, buf)
            return carry

        lax.fori_loop(0, n_sc, body, 0)
        wwait(lax.rem(n_sc - 1, 2))

    return k(idx3, table)


def kernel(indices, embedding):
    b, s = indices.shape
    s_pad = (s + 7) // 8 * 8
    idx = indices.astype(jnp.int32)
    idx = jnp.pad(idx, ((0, 0), (0, s_pad - s)))
    idx3 = idx.reshape(NUM_WORKERS, b // NUM_WORKERS, s_pad)
    return _gather_sc(idx3, embedding, b, s, s_pad)
